# baseline (speedup 1.0000x reference)
import functools
import jax
import jax.numpy as jnp
from jax import lax
from jax.experimental import pallas as pl
from jax.experimental.pallas import tpu as pltpu
from jax.experimental.pallas import tpu_sc as plsc

_D = 64
_NC = 2
_NS = 16
_NW = _NC * _NS
_LANES = 16
_BB = 128


def _emb_lookup(x_t, table, B, T):
  assert B == _NW * _BB

  mesh = plsc.VectorSubcoreMesh(
      core_axis_name="c", subcore_axis_name="s",
      num_cores=_NC, num_subcores=_NS)

  @functools.partial(
      pl.kernel,
      out_type=jax.ShapeDtypeStruct((T, _D // 8, B // _BB, 8, _BB),
                                    jnp.float32),
      mesh=mesh,
      compiler_params=pltpu.CompilerParams(use_tc_tiling_on_sc=False,
                                           needs_layout_passes=False),
      scratch_types=[
          pltpu.VMEM((T, _BB), jnp.int32),
          pltpu.VMEM((2, _BB, _D), jnp.float32),
          pltpu.VMEM((2, 8, 8, _BB + 1), jnp.float32),
          pltpu.SemaphoreType.DMA,
          pltpu.SemaphoreType.DMA,
          pltpu.SemaphoreType.DMA,
          pltpu.SemaphoreType.DMA,
      ],
  )
  def k(xt_hbm, table_hbm, out_hbm, idx_v, rows_v, tiles_v,
        sem0, sem1, osem0, osem1):
    wid = lax.axis_index("s") * _NC + lax.axis_index("c")
    b0 = pl.multiple_of(wid * _BB, _BB)
    pltpu.sync_copy(xt_hbm.at[:, pl.ds(b0, _BB)], idx_v)

    sems = (sem0, sem1)

    def start_gather(t, b):
      pltpu.async_copy(table_hbm.at[idx_v.at[t]], rows_v.at[b], sems[b])

    start_gather(0, 0)
    start_gather(1, 1)

    iota = lax.iota(jnp.int32, _LANES)
    js_g = [lax.shift_right_logical(iota, 3) + 2 * g for g in range(4)]
    jr = lax.bitwise_and(iota, 7)
    osems = (osem0, osem1)

    def pair_body(p, _):
      for b in range(2):
        t = p * 2 + b
        buf = rows_v.at[b]
        st = tiles_v.at[b]
        pltpu.make_async_copy(table_hbm.at[pl.ds(0, _BB)],
                              buf, sems[b]).wait()
        @pl.when(t >= 2)
        def _():
          pltpu.make_async_copy(out_hbm.at[0, :, 0],
                                st.at[:, :, pl.ds(0, _BB)],
                                osems[b]).wait()

        @plsc.parallel_loop(0, _BB, 1, unroll=8)
        def _permute(bl):
          bl_s = jnp.broadcast_to(bl, (_LANES,))
          vs = [buf[bl, pl.ds(16 * g, _LANES)] * 8.0 for g in range(4)]
          for g in range(4):
            plsc.store_scatter(st, [js_g[g], jr, bl_s], vs[g])

        pltpu.async_copy(st.at[:, :, pl.ds(0, _BB)],
                         out_hbm.at[t, :, wid], osems[b])

        @pl.when(t + 2 < T)
        def _():
          start_gather(t + 2, b)
      return ()

    lax.fori_loop(0, T // 2, pair_body, ())
    for b in range(2):
      pltpu.make_async_copy(out_hbm.at[0, :, 0],
                            tiles_v.at[b, :, :, pl.ds(0, _BB)],
                            osems[b]).wait()

  return k(x_t, table)


def kernel(x, emb_weight):
  b, t = x.shape
  x_t = x.T
  th = t // 2
  out5a = _emb_lookup(x_t[:th], emb_weight, b, th)
  out5b = _emb_lookup(x_t[th:], emb_weight, b, t - th)
  out5 = jnp.concatenate([out5a, out5b], axis=0)
  out = out5.transpose(2, 4, 0, 1, 3).reshape(b, t, _D)
  return out


# speedup vs baseline: 1.2551x; 1.2551x over previous
import functools
import jax
import jax.numpy as jnp
from jax import lax
from jax.experimental import pallas as pl
from jax.experimental.pallas import tpu as pltpu
from jax.experimental.pallas import tpu_sc as plsc

_D = 64
_NC = 2
_NS = 16
_NW = _NC * _NS
_LANES = 16
_BB = 128


def _emb_lookup(x_t, table, B, T):
  assert B == _NW * _BB

  mesh = plsc.VectorSubcoreMesh(
      core_axis_name="c", subcore_axis_name="s",
      num_cores=_NC, num_subcores=_NS)

  @functools.partial(
      pl.kernel,
      out_type=jax.ShapeDtypeStruct((T, _D // 8, B // _BB, 8, _BB),
                                    jnp.float32),
      mesh=mesh,
      compiler_params=pltpu.CompilerParams(use_tc_tiling_on_sc=False,
                                           needs_layout_passes=False),
      scratch_types=[
          pltpu.VMEM((T, _BB), jnp.int32),
          pltpu.VMEM((2, 2 * _BB, _D), jnp.float32),
          pltpu.VMEM((2, 2, 8, 8, _BB + 1), jnp.float32),
          pltpu.SemaphoreType.DMA,
          pltpu.SemaphoreType.DMA,
          pltpu.SemaphoreType.DMA,
          pltpu.SemaphoreType.DMA,
      ],
  )
  def k(xt_hbm, table_hbm, out_hbm, idx_v, rows_v, tiles_v,
        sem0, sem1, osem0, osem1):
    wid = lax.axis_index("s") * _NC + lax.axis_index("c")
    b0 = pl.multiple_of(wid * _BB, _BB)
    pltpu.sync_copy(xt_hbm.at[:, pl.ds(b0, _BB)], idx_v)

    sems = (sem0, sem1)

    def start_gather(c, b):
      t = pl.multiple_of(2 * c, 2)
      pltpu.async_copy(table_hbm.at[idx_v.at[t]],
                       rows_v.at[b, pl.ds(0, _BB)], sems[b])
      pltpu.async_copy(table_hbm.at[idx_v.at[t + 1]],
                       rows_v.at[b, pl.ds(_BB, _BB)], sems[b])

    start_gather(0, 0)
    start_gather(1, 1)

    iota = lax.iota(jnp.int32, _LANES)
    js_g = [lax.shift_right_logical(iota, 3) + 2 * g for g in range(4)]
    jr = lax.bitwise_and(iota, 7)
    osems = (osem0, osem1)

    def pair_body(p, _):
      for b in range(2):
        c = p * 2 + b
        t = pl.multiple_of(2 * c, 2)
        buf = rows_v.at[b]
        st = tiles_v.at[b]
        for _i in range(2):
          pltpu.make_async_copy(table_hbm.at[pl.ds(0, _BB)],
                                buf.at[pl.ds(0, _BB)], sems[b]).wait()
        @pl.when(c >= 2)
        def _():
          pltpu.make_async_copy(out_hbm.at[pl.ds(0, 2), :, 0],
                                st.at[:, :, :, pl.ds(0, _BB)],
                                osems[b]).wait()

        @plsc.parallel_loop(0, 2 * _BB, 1, unroll=8)
        def _permute(r):
          tt_s = jnp.broadcast_to(lax.shift_right_logical(r, 7), (_LANES,))
          bl_s = jnp.broadcast_to(lax.bitwise_and(r, _BB - 1), (_LANES,))
          vs = [buf[r, pl.ds(16 * g, _LANES)] * 8.0 for g in range(4)]
          for g in range(4):
            plsc.store_scatter(st, [tt_s, js_g[g], jr, bl_s], vs[g])

        pltpu.async_copy(st.at[:, :, :, pl.ds(0, _BB)],
                         out_hbm.at[pl.ds(t, 2), :, wid], osems[b])

        @pl.when(c + 2 < T // 2)
        def _():
          start_gather(c + 2, b)
      return ()

    lax.fori_loop(0, T // 4, pair_body, ())
    for b in range(2):
      pltpu.make_async_copy(out_hbm.at[pl.ds(0, 2), :, 0],
                            tiles_v.at[b, :, :, :, pl.ds(0, _BB)],
                            osems[b]).wait()

  return k(x_t, table)


def kernel(x, emb_weight):
  b, t = x.shape
  out5 = _emb_lookup(x.T, emb_weight, b, t)
  out = out5.transpose(2, 4, 0, 1, 3).reshape(b, t, _D)
  return out
